# baseline (speedup 1.0000x reference)
import functools

import jax
import jax.numpy as jnp
from jax import lax
from jax.experimental import pallas as pl
from jax.experimental.pallas import tpu as pltpu
from jax.experimental.pallas import tpu_sc as plsc

N_NODES = 10000
N_EDGES = 160000
D = 256
HD = 128
N_CLS = 10

NC = 2
NS = 16
LANES = 16

EDGES_PER_TILE = N_EDGES // NS
CHUNK = 128
N_FULL = EDGES_PER_TILE // CHUNK
REM = EDGES_PER_TILE - N_FULL * CHUNK
ROWS_PER_TILE = N_NODES // NS

HALF = EDGES_PER_TILE // 2
HCH = HALF // CHUNK
HREM = HALF - HCH * CHUNK
ACC_ROWS = N_NODES + 8

_sc_mesh = plsc.VectorSubcoreMesh(
    core_axis_name="c", subcore_axis_name="s", num_cores=NC, num_subcores=NS)


RO_ROWS = 624
RO_LAST = N_NODES - RO_ROWS * (NS - 1)


def _readout(shared_ref, out_hbm, c, s):
    row8 = pl.multiple_of(s * RO_ROWS, 8)

    @pl.when(s < NS - 1)
    def _():
        pltpu.sync_copy(shared_ref.at[pl.ds(row8, RO_ROWS)],
                        out_hbm.at[c, pl.ds(row8, RO_ROWS)])

    @pl.when(s == NS - 1)
    def _():
        pltpu.sync_copy(shared_ref.at[pl.ds(row8, RO_LAST)],
                        out_hbm.at[c, pl.ds(row8, RO_LAST)])


def _zero_vmem_rows(ref, nrows, ncols):
    zv = jnp.zeros((LANES,), jnp.float32)
    for j in range(ncols // LANES):
        def body(i, _, j=j):
            ref[i, pl.ds(j * LANES, LANES)] = zv
            return 0
        lax.fori_loop(0, nrows, body, 0)


@functools.partial(
    pl.kernel,
    out_type=jax.ShapeDtypeStruct((NC, N_NODES, HD), jnp.float32),
    mesh=_sc_mesh,
    scratch_types=[
        pltpu.VMEM_SHARED((N_NODES, HD), jnp.float32),
        pltpu.VMEM((CHUNK, HD), jnp.float32),
        pltpu.VMEM((EDGES_PER_TILE,), jnp.int32),
        pltpu.VMEM((N_FULL, CHUNK), jnp.int32),
        pltpu.VMEM((REM,), jnp.int32),
        pltpu.SemaphoreType.DMA,
    ],
)
def _sc_degrees(src_hbm, dst_hbm, deg_hbm,
                hist, ones_v, draw, dst2d, didx_t, sem):
    c = lax.axis_index("c")
    s = lax.axis_index("s")

    _zero_vmem_rows(ones_v, 125, HD)
    row0 = s * ROWS_PER_TILE
    for k in range(5):
        pltpu.sync_copy(ones_v.at[pl.ds(0, 125)],
                        hist.at[pl.ds(row0 + k * 125, 125)])

    onev = jnp.ones((LANES,), jnp.float32)

    def fill_ones(i, _):
        for j in range(HD // LANES):
            ones_v[i, pl.ds(j * LANES, LANES)] = onev
        return 0
    lax.fori_loop(0, CHUNK, fill_ones, 0)

    base0 = s * EDGES_PER_TILE

    @pl.when(c == 0)
    def _():
        pltpu.sync_copy(src_hbm.at[pl.ds(base0, EDGES_PER_TILE)], draw)

    @pl.when(c == 1)
    def _():
        pltpu.sync_copy(dst_hbm.at[pl.ds(base0, EDGES_PER_TILE)], draw)

    def dfill(r, _):
        for j in range(HD // LANES):
            dst2d[r, pl.ds(j * LANES, LANES)] = \
                draw[pl.ds(r * CHUNK + j * LANES, LANES)]
        return 0
    lax.fori_loop(0, N_FULL, dfill, 0)
    didx_t[pl.ds(0, REM)] = draw[pl.ds(N_FULL * CHUNK, REM)]
    plsc.subcore_barrier()

    def fire6(r6, _):
        ds_ = [pltpu.async_copy(ones_v, hist.at[dst2d.at[r6 * 6 + j]], sem,
                                add=True)
               for j in range(6)]
        for d in ds_:
            d.wait()
        return 0
    lax.fori_loop(0, N_FULL // 6, fire6, 0)
    pltpu.sync_copy(ones_v.at[pl.ds(0, REM)], hist.at[didx_t], add=True)

    plsc.subcore_barrier()
    _readout(hist, deg_hbm, c, s)


@functools.partial(
    pl.kernel,
    out_type=jax.ShapeDtypeStruct((NC, N_NODES, HD), jnp.float32),
    mesh=_sc_mesh,
    scratch_types=[
        pltpu.VMEM_SHARED((ACC_ROWS, HD), jnp.float32),
        pltpu.VMEM((CHUNK, HD), jnp.float32),
        pltpu.VMEM((CHUNK, HD), jnp.float32),
        pltpu.VMEM((HALF + 8,), jnp.int32),
        pltpu.VMEM((HALF + 8,), jnp.int32),
        pltpu.VMEM((HCH, CHUNK), jnp.int32),
        pltpu.VMEM((LANES,), jnp.int32),
        pltpu.SemaphoreType.DMA,
        pltpu.SemaphoreType.DMA,
    ],
)
def _sc_aggregate(x2_hbm, src_hbm, dst_hbm, agg_hbm,
                  acc, rows_a, rows_b, sidx, draw, dst2d, didx_t,
                  sem_a, sem_b):
    c = lax.axis_index("c")
    s = lax.axis_index("s")

    _zero_vmem_rows(rows_a, 125, HD)
    row0 = s * ROWS_PER_TILE
    for k in range(5):
        pltpu.sync_copy(rows_a.at[pl.ds(0, 125)],
                        acc.at[pl.ds(row0 + k * 125, 125)])
    plsc.subcore_barrier()

    GSPLIT = 4
    GS = CHUNK // GSPLIT

    def gather(r, buf, sem):
        for g in range(GSPLIT):
            pltpu.async_copy(
                x2_hbm.at[sidx.at[pl.ds(r * CHUNK + g * GS, GS)]],
                buf.at[pl.ds(g * GS, GS)], sem)

    def gwait(r, buf, sem):
        for g in range(GSPLIT):
            pltpu.make_async_copy(
                x2_hbm.at[sidx.at[pl.ds(r * CHUNK + g * GS, GS)]],
                buf.at[pl.ds(g * GS, GS)], sem).wait()

    def scatter(r, buf):
        pltpu.sync_copy(buf, acc.at[dst2d.at[r]], add=True)

    lane = lax.iota(jnp.int32, LANES)

    for h in range(2):
        base = s * EDGES_PER_TILE + h * HALF

        pltpu.sync_copy(src_hbm.at[pl.ds(base, HALF)], sidx.at[pl.ds(0, HALF)])
        pltpu.sync_copy(dst_hbm.at[pl.ds(base, HALF)], draw.at[pl.ds(0, HALF)])

        def xform(i, _):
            v = sidx[pl.ds(i * LANES, LANES)]
            sidx[pl.ds(i * LANES, LANES)] = v + v + c
            return 0
        lax.fori_loop(0, HALF // LANES, xform, 0)
        v = sidx[pl.ds(HCH * CHUNK, LANES)]
        sidx[pl.ds(HCH * CHUNK, LANES)] = \
            jnp.where(lane < HREM, v + v + c, 0)
        d = draw[pl.ds(HCH * CHUNK, LANES)]
        didx_t[pl.ds(0, LANES)] = jnp.where(lane < HREM, d, N_NODES)

        def dfill(r, _):
            for j in range(CHUNK // LANES):
                dst2d[r, pl.ds(j * LANES, LANES)] = \
                    draw[pl.ds(r * CHUNK + j * LANES, LANES)]
            return 0
        lax.fori_loop(0, HCH, dfill, 0)

        pltpu.async_copy(x2_hbm.at[sidx.at[pl.ds(HCH * CHUNK, LANES)]],
                         rows_a.at[pl.ds(0, LANES)], sem_a).wait()
        pltpu.sync_copy(rows_a.at[pl.ds(0, LANES)], acc.at[didx_t], add=True)
        gather(0, rows_a, sem_a)
        gwait(0, rows_a, sem_a)
        scatter(0, rows_a)

        gather(1, rows_a, sem_a)

        def body(r2, _):
            e = r2 * 2 + 1
            gwait(e, rows_a, sem_a)
            gather(e + 1, rows_b, sem_b)
            scatter(e, rows_a)
            gwait(e + 1, rows_b, sem_b)
            @pl.when(r2 < (HCH - 1) // 2 - 1)
            def _():
                gather(e + 2, rows_a, sem_a)
            scatter(e + 1, rows_b)
            return 0
        lax.fori_loop(0, (HCH - 1) // 2, body, 0)

    plsc.subcore_barrier()
    _readout(acc, agg_hbm, c, s)


ROWS_BLK = 1000
N_BLK = N_NODES // ROWS_BLK


def _elu(x):
    return jnp.where(x > 0, x, jnp.exp(x) - 1.0)


def _tc_prep_body(h_ref, od_ref, id_ref, x0_ref, ns_ref, nd_ref):
    od = od_ref[...]
    idg = id_ref[...]
    ns = lax.rsqrt(jnp.maximum(od, 1.0))
    nd = lax.rsqrt(jnp.maximum(idg, 1.0))
    ns_ref[...] = ns
    nd_ref[...] = nd
    x0_ref[...] = h_ref[...] * ns


def _tc_prep(h, out_deg, in_deg):
    return pl.pallas_call(
        _tc_prep_body,
        grid=(N_BLK,),
        in_specs=[
            pl.BlockSpec((ROWS_BLK, D), lambda i: (i, 0)),
            pl.BlockSpec((ROWS_BLK, 1), lambda i: (i, 0)),
            pl.BlockSpec((ROWS_BLK, 1), lambda i: (i, 0)),
        ],
        out_specs=[
            pl.BlockSpec((ROWS_BLK, D), lambda i: (i, 0)),
            pl.BlockSpec((ROWS_BLK, 1), lambda i: (i, 0)),
            pl.BlockSpec((ROWS_BLK, 1), lambda i: (i, 0)),
        ],
        out_shape=[
            jax.ShapeDtypeStruct((N_NODES, D), jnp.float32),
            jax.ShapeDtypeStruct((N_NODES, 1), jnp.float32),
            jax.ShapeDtypeStruct((N_NODES, 1), jnp.float32),
        ],
    )(h, out_deg, in_deg)


def _tc_layer_body(lo_ref, hi_ref, nd_ref, ns_ref, w_ref, b_ref, out_ref):
    nd = nd_ref[...]
    a_lo = lo_ref[0] * nd
    a_hi = hi_ref[0] * nd
    y = (jnp.dot(a_lo, w_ref[0:HD, :], preferred_element_type=jnp.float32)
         + jnp.dot(a_hi, w_ref[HD:D, :], preferred_element_type=jnp.float32)
         + b_ref[...])
    out_ref[...] = _elu(y) * ns_ref[...]


def _tc_layer(agg3, norm_dst, norm_src, w, b):
    return pl.pallas_call(
        _tc_layer_body,
        grid=(N_BLK,),
        in_specs=[
            pl.BlockSpec((1, ROWS_BLK, HD), lambda i: (0, i, 0)),
            pl.BlockSpec((1, ROWS_BLK, HD), lambda i: (1, i, 0)),
            pl.BlockSpec((ROWS_BLK, 1), lambda i: (i, 0)),
            pl.BlockSpec((ROWS_BLK, 1), lambda i: (i, 0)),
            pl.BlockSpec((D, D), lambda i: (0, 0)),
            pl.BlockSpec((1, D), lambda i: (0, 0)),
        ],
        out_specs=pl.BlockSpec((ROWS_BLK, D), lambda i: (i, 0)),
        out_shape=jax.ShapeDtypeStruct((N_NODES, D), jnp.float32),
    )(agg3, agg3, norm_dst, norm_src, w, b.reshape(1, D))


def _tc_final_body(lo_ref, hi_ref, nd_ref, w_ref, b_ref,
                   f1w_ref, f1b_ref, f2w_ref, f2b_ref, out_ref, acc_ref):
    i = pl.program_id(0)
    nd = nd_ref[...]
    a_lo = lo_ref[0] * nd
    a_hi = hi_ref[0] * nd
    y = (jnp.dot(a_lo, w_ref[0:HD, :], preferred_element_type=jnp.float32)
         + jnp.dot(a_hi, w_ref[HD:D, :], preferred_element_type=jnp.float32)
         + b_ref[...])
    y = _elu(y)
    blk = jnp.sum(y, axis=0, keepdims=True)

    @pl.when(i == 0)
    def _():
        acc_ref[...] = blk

    @pl.when(i > 0)
    def _():
        acc_ref[...] = acc_ref[...] + blk

    @pl.when(i == N_BLK - 1)
    def _():
        g = acc_ref[...] * (1.0 / N_NODES)
        g1 = _elu(jnp.dot(g, f1w_ref[...], preferred_element_type=jnp.float32)
                  + f1b_ref[...])
        g2 = (jnp.dot(g1, f2w_ref[...], preferred_element_type=jnp.float32)
              + f2b_ref[...])
        m = jnp.max(g2, axis=1, keepdims=True)
        lse = m + jnp.log(jnp.sum(jnp.exp(g2 - m), axis=1, keepdims=True))
        out_ref[...] = g2 - lse


def _tc_final(agg3, norm_dst, w, b, f1w, f1b, f2w, f2b):
    return pl.pallas_call(
        _tc_final_body,
        grid=(N_BLK,),
        in_specs=[
            pl.BlockSpec((1, ROWS_BLK, HD), lambda i: (0, i, 0)),
            pl.BlockSpec((1, ROWS_BLK, HD), lambda i: (1, i, 0)),
            pl.BlockSpec((ROWS_BLK, 1), lambda i: (i, 0)),
            pl.BlockSpec((D, D), lambda i: (0, 0)),
            pl.BlockSpec((1, D), lambda i: (0, 0)),
            pl.BlockSpec((D, D), lambda i: (0, 0)),
            pl.BlockSpec((1, D), lambda i: (0, 0)),
            pl.BlockSpec((D, N_CLS), lambda i: (0, 0)),
            pl.BlockSpec((1, N_CLS), lambda i: (0, 0)),
        ],
        out_specs=pl.BlockSpec((1, N_CLS), lambda i: (0, 0)),
        out_shape=jax.ShapeDtypeStruct((1, N_CLS), jnp.float32),
        scratch_shapes=[pltpu.VMEM((1, D), jnp.float32)],
    )(agg3, agg3, norm_dst, w, b.reshape(1, D),
      f1w, f1b.reshape(1, D), f2w, f2b.reshape(1, N_CLS))


def kernel(h, edge_index, W0, b0, W1, b1, W2, b2, W3, b3,
           fc1_W, fc1_b, fc2_W, fc2_b):
    src = edge_index[0]
    dst = edge_index[1]
    deg2 = _sc_degrees(src, dst)
    out_deg = deg2[0, :, 0:1]
    in_deg = deg2[1, :, 0:1]

    x, norm_src, norm_dst = _tc_prep(h, out_deg, in_deg)

    for (w, b) in ((W0, b0), (W1, b1), (W2, b2)):
        agg3 = _sc_aggregate(x.reshape(2 * N_NODES, HD), src, dst)
        x = _tc_layer(agg3, norm_dst, norm_src, w, b)

    agg3 = _sc_aggregate(x.reshape(2 * N_NODES, HD), src, dst)
    return _tc_final(agg3, norm_dst, W3, b3, fc1_W, fc1_b, fc2_W, fc2_b)


# speedup vs baseline: 1.0108x; 1.0108x over previous
import functools

import jax
import jax.numpy as jnp
from jax import lax
from jax.experimental import pallas as pl
from jax.experimental.pallas import tpu as pltpu
from jax.experimental.pallas import tpu_sc as plsc

N_NODES = 10000
N_EDGES = 160000
D = 256
HD = 128
N_CLS = 10

NC = 2
NS = 16
LANES = 16

EDGES_PER_TILE = N_EDGES // NS
CHUNK = 128
N_FULL = EDGES_PER_TILE // CHUNK
REM = EDGES_PER_TILE - N_FULL * CHUNK
ROWS_PER_TILE = N_NODES // NS

HALF = EDGES_PER_TILE // 2
HCH = HALF // CHUNK
HREM = HALF - HCH * CHUNK
ACC_ROWS = N_NODES + 8

_sc_mesh = plsc.VectorSubcoreMesh(
    core_axis_name="c", subcore_axis_name="s", num_cores=NC, num_subcores=NS)


RO_ROWS = 624
RO_LAST = N_NODES - RO_ROWS * (NS - 1)


def _readout(shared_ref, out_hbm, c, s):
    row8 = pl.multiple_of(s * RO_ROWS, 8)

    @pl.when(s < NS - 1)
    def _():
        pltpu.sync_copy(shared_ref.at[pl.ds(row8, RO_ROWS)],
                        out_hbm.at[c, pl.ds(row8, RO_ROWS)])

    @pl.when(s == NS - 1)
    def _():
        pltpu.sync_copy(shared_ref.at[pl.ds(row8, RO_LAST)],
                        out_hbm.at[c, pl.ds(row8, RO_LAST)])


def _zero_vmem_rows(ref, nrows, ncols):
    zv = jnp.zeros((LANES,), jnp.float32)
    for j in range(ncols // LANES):
        def body(i, _, j=j):
            ref[i, pl.ds(j * LANES, LANES)] = zv
            return 0
        lax.fori_loop(0, nrows, body, 0)


@functools.partial(
    pl.kernel,
    out_type=jax.ShapeDtypeStruct((NC, N_NODES, HD), jnp.float32),
    mesh=_sc_mesh,
    scratch_types=[
        pltpu.VMEM_SHARED((N_NODES, HD), jnp.float32),
        pltpu.VMEM((CHUNK, HD), jnp.float32),
        pltpu.VMEM((EDGES_PER_TILE,), jnp.int32),
        pltpu.VMEM((N_FULL, CHUNK), jnp.int32),
        pltpu.VMEM((REM,), jnp.int32),
        pltpu.SemaphoreType.DMA,
    ],
)
def _sc_degrees(src_hbm, dst_hbm, deg_hbm,
                hist, ones_v, draw, dst2d, didx_t, sem):
    c = lax.axis_index("c")
    s = lax.axis_index("s")

    _zero_vmem_rows(ones_v, 125, HD)
    row0 = s * ROWS_PER_TILE
    for k in range(5):
        pltpu.sync_copy(ones_v.at[pl.ds(0, 125)],
                        hist.at[pl.ds(row0 + k * 125, 125)])

    onev = jnp.ones((LANES,), jnp.float32)

    def fill_ones(i, _):
        for j in range(HD // LANES):
            ones_v[i, pl.ds(j * LANES, LANES)] = onev
        return 0
    lax.fori_loop(0, CHUNK, fill_ones, 0)

    base0 = s * EDGES_PER_TILE

    @pl.when(c == 0)
    def _():
        pltpu.sync_copy(src_hbm.at[pl.ds(base0, EDGES_PER_TILE)], draw)

    @pl.when(c == 1)
    def _():
        pltpu.sync_copy(dst_hbm.at[pl.ds(base0, EDGES_PER_TILE)], draw)

    def dfill(r, _):
        for j in range(HD // LANES):
            dst2d[r, pl.ds(j * LANES, LANES)] = \
                draw[pl.ds(r * CHUNK + j * LANES, LANES)]
        return 0
    lax.fori_loop(0, N_FULL, dfill, 0)
    didx_t[pl.ds(0, REM)] = draw[pl.ds(N_FULL * CHUNK, REM)]
    plsc.subcore_barrier()

    def fire6(r6, _):
        ds_ = [pltpu.async_copy(ones_v, hist.at[dst2d.at[r6 * 6 + j]], sem,
                                add=True)
               for j in range(6)]
        for d in ds_:
            d.wait()
        return 0
    lax.fori_loop(0, N_FULL // 6, fire6, 0)
    pltpu.sync_copy(ones_v.at[pl.ds(0, REM)], hist.at[didx_t], add=True)

    plsc.subcore_barrier()
    _readout(hist, deg_hbm, c, s)


@functools.partial(
    pl.kernel,
    out_type=jax.ShapeDtypeStruct((NC, N_NODES, HD), jnp.float32),
    mesh=_sc_mesh,
    scratch_types=[
        pltpu.VMEM_SHARED((ACC_ROWS, HD), jnp.float32),
        pltpu.VMEM((CHUNK, HD), jnp.float32),
        pltpu.VMEM((CHUNK, HD), jnp.float32),
        pltpu.VMEM((HALF + 8,), jnp.int32),
        pltpu.VMEM((HALF + 8,), jnp.int32),
        pltpu.VMEM((HCH, CHUNK), jnp.int32),
        pltpu.VMEM((LANES,), jnp.int32),
        pltpu.SemaphoreType.DMA,
        pltpu.SemaphoreType.DMA,
    ],
)
def _sc_aggregate(x2_hbm, src_hbm, dst_hbm, agg_hbm,
                  acc, rows_a, rows_b, sidx, draw, dst2d, didx_t,
                  sem_a, sem_b):
    c = lax.axis_index("c")
    s = lax.axis_index("s")
    lane = lax.iota(jnp.int32, LANES)
    row0 = s * ROWS_PER_TILE

    def gather(r, buf, sem):
        pltpu.async_copy(
            x2_hbm.at[sidx.at[pl.ds(r * CHUNK, CHUNK)]], buf, sem)

    def gwait(r, buf, sem):
        pltpu.make_async_copy(
            x2_hbm.at[sidx.at[pl.ds(r * CHUNK, CHUNK)]], buf, sem).wait()

    def scatter(r, buf):
        pltpu.sync_copy(buf, acc.at[dst2d.at[r]], add=True)

    def stage(h):
        base = s * EDGES_PER_TILE + h * HALF
        pltpu.sync_copy(src_hbm.at[pl.ds(base, HALF)], sidx.at[pl.ds(0, HALF)])
        pltpu.sync_copy(dst_hbm.at[pl.ds(base, HALF)], draw.at[pl.ds(0, HALF)])

        def xform(i, _):
            v = sidx[pl.ds(i * LANES, LANES)]
            sidx[pl.ds(i * LANES, LANES)] = v + v + c
            return 0
        lax.fori_loop(0, HALF // LANES, xform, 0)
        v = sidx[pl.ds(HCH * CHUNK, LANES)]
        sidx[pl.ds(HCH * CHUNK, LANES)] = \
            jnp.where(lane < HREM, v + v + c, 0)
        d = draw[pl.ds(HCH * CHUNK, LANES)]
        didx_t[pl.ds(0, LANES)] = jnp.where(lane < HREM, d, N_NODES)

        def dfill(r, _):
            for j in range(CHUNK // LANES):
                dst2d[r, pl.ds(j * LANES, LANES)] = \
                    draw[pl.ds(r * CHUNK + j * LANES, LANES)]
            return 0
        lax.fori_loop(0, HCH, dfill, 0)

    def run_half():
        pltpu.async_copy(x2_hbm.at[sidx.at[pl.ds(HCH * CHUNK, LANES)]],
                         rows_a.at[pl.ds(0, LANES)], sem_a).wait()
        pltpu.sync_copy(rows_a.at[pl.ds(0, LANES)], acc.at[didx_t], add=True)
        gather(0, rows_a, sem_a)
        gwait(0, rows_a, sem_a)
        scatter(0, rows_a)

        gather(1, rows_a, sem_a)

        def body(r2, _):
            e = r2 * 2 + 1
            gwait(e, rows_a, sem_a)
            gather(e + 1, rows_b, sem_b)
            scatter(e, rows_a)
            gwait(e + 1, rows_b, sem_b)
            @pl.when(r2 < (HCH - 1) // 2 - 1)
            def _():
                gather(e + 2, rows_a, sem_a)
            scatter(e + 1, rows_b)
            return 0
        lax.fori_loop(0, (HCH - 1) // 2, body, 0)

    _zero_vmem_rows(rows_a, 125, HD)
    for k in range(5):
        pltpu.async_copy(rows_a.at[pl.ds(0, 125)],
                         acc.at[pl.ds(row0 + k * 125, 125)], sem_b)
    stage(0)
    for k in range(5):
        pltpu.make_async_copy(rows_a.at[pl.ds(0, 125)],
                              acc.at[pl.ds(row0 + k * 125, 125)],
                              sem_b).wait()
    plsc.subcore_barrier()

    run_half()
    stage(1)
    run_half()

    plsc.subcore_barrier()
    _readout(acc, agg_hbm, c, s)


ROWS_BLK = 1000
N_BLK = N_NODES // ROWS_BLK


def _elu(x):
    return jnp.where(x > 0, x, jnp.exp(x) - 1.0)


def _tc_prep_body(h_ref, od_ref, id_ref, x0_ref, ns_ref, nd_ref):
    od = od_ref[...]
    idg = id_ref[...]
    ns = lax.rsqrt(jnp.maximum(od, 1.0))
    nd = lax.rsqrt(jnp.maximum(idg, 1.0))
    ns_ref[...] = ns
    nd_ref[...] = nd
    x0_ref[...] = h_ref[...] * ns


def _tc_prep(h, out_deg, in_deg):
    return pl.pallas_call(
        _tc_prep_body,
        grid=(N_BLK,),
        in_specs=[
            pl.BlockSpec((ROWS_BLK, D), lambda i: (i, 0)),
            pl.BlockSpec((ROWS_BLK, 1), lambda i: (i, 0)),
            pl.BlockSpec((ROWS_BLK, 1), lambda i: (i, 0)),
        ],
        out_specs=[
            pl.BlockSpec((ROWS_BLK, D), lambda i: (i, 0)),
            pl.BlockSpec((ROWS_BLK, 1), lambda i: (i, 0)),
            pl.BlockSpec((ROWS_BLK, 1), lambda i: (i, 0)),
        ],
        out_shape=[
            jax.ShapeDtypeStruct((N_NODES, D), jnp.float32),
            jax.ShapeDtypeStruct((N_NODES, 1), jnp.float32),
            jax.ShapeDtypeStruct((N_NODES, 1), jnp.float32),
        ],
    )(h, out_deg, in_deg)


def _tc_layer_body(lo_ref, hi_ref, nd_ref, ns_ref, w_ref, b_ref, out_ref):
    nd = nd_ref[...]
    a_lo = lo_ref[0] * nd
    a_hi = hi_ref[0] * nd
    y = (jnp.dot(a_lo, w_ref[0:HD, :], preferred_element_type=jnp.float32)
         + jnp.dot(a_hi, w_ref[HD:D, :], preferred_element_type=jnp.float32)
         + b_ref[...])
    out_ref[...] = _elu(y) * ns_ref[...]


def _tc_layer(agg3, norm_dst, norm_src, w, b):
    return pl.pallas_call(
        _tc_layer_body,
        grid=(N_BLK,),
        in_specs=[
            pl.BlockSpec((1, ROWS_BLK, HD), lambda i: (0, i, 0)),
            pl.BlockSpec((1, ROWS_BLK, HD), lambda i: (1, i, 0)),
            pl.BlockSpec((ROWS_BLK, 1), lambda i: (i, 0)),
            pl.BlockSpec((ROWS_BLK, 1), lambda i: (i, 0)),
            pl.BlockSpec((D, D), lambda i: (0, 0)),
            pl.BlockSpec((1, D), lambda i: (0, 0)),
        ],
        out_specs=pl.BlockSpec((ROWS_BLK, D), lambda i: (i, 0)),
        out_shape=jax.ShapeDtypeStruct((N_NODES, D), jnp.float32),
    )(agg3, agg3, norm_dst, norm_src, w, b.reshape(1, D))


def _tc_final_body(lo_ref, hi_ref, nd_ref, w_ref, b_ref,
                   f1w_ref, f1b_ref, f2w_ref, f2b_ref, out_ref, acc_ref):
    i = pl.program_id(0)
    nd = nd_ref[...]
    a_lo = lo_ref[0] * nd
    a_hi = hi_ref[0] * nd
    y = (jnp.dot(a_lo, w_ref[0:HD, :], preferred_element_type=jnp.float32)
         + jnp.dot(a_hi, w_ref[HD:D, :], preferred_element_type=jnp.float32)
         + b_ref[...])
    y = _elu(y)
    blk = jnp.sum(y, axis=0, keepdims=True)

    @pl.when(i == 0)
    def _():
        acc_ref[...] = blk

    @pl.when(i > 0)
    def _():
        acc_ref[...] = acc_ref[...] + blk

    @pl.when(i == N_BLK - 1)
    def _():
        g = acc_ref[...] * (1.0 / N_NODES)
        g1 = _elu(jnp.dot(g, f1w_ref[...], preferred_element_type=jnp.float32)
                  + f1b_ref[...])
        g2 = (jnp.dot(g1, f2w_ref[...], preferred_element_type=jnp.float32)
              + f2b_ref[...])
        m = jnp.max(g2, axis=1, keepdims=True)
        lse = m + jnp.log(jnp.sum(jnp.exp(g2 - m), axis=1, keepdims=True))
        out_ref[...] = g2 - lse


def _tc_final(agg3, norm_dst, w, b, f1w, f1b, f2w, f2b):
    return pl.pallas_call(
        _tc_final_body,
        grid=(N_BLK,),
        in_specs=[
            pl.BlockSpec((1, ROWS_BLK, HD), lambda i: (0, i, 0)),
            pl.BlockSpec((1, ROWS_BLK, HD), lambda i: (1, i, 0)),
            pl.BlockSpec((ROWS_BLK, 1), lambda i: (i, 0)),
            pl.BlockSpec((D, D), lambda i: (0, 0)),
            pl.BlockSpec((1, D), lambda i: (0, 0)),
            pl.BlockSpec((D, D), lambda i: (0, 0)),
            pl.BlockSpec((1, D), lambda i: (0, 0)),
            pl.BlockSpec((D, N_CLS), lambda i: (0, 0)),
            pl.BlockSpec((1, N_CLS), lambda i: (0, 0)),
        ],
        out_specs=pl.BlockSpec((1, N_CLS), lambda i: (0, 0)),
        out_shape=jax.ShapeDtypeStruct((1, N_CLS), jnp.float32),
        scratch_shapes=[pltpu.VMEM((1, D), jnp.float32)],
    )(agg3, agg3, norm_dst, w, b.reshape(1, D),
      f1w, f1b.reshape(1, D), f2w, f2b.reshape(1, N_CLS))


def kernel(h, edge_index, W0, b0, W1, b1, W2, b2, W3, b3,
           fc1_W, fc1_b, fc2_W, fc2_b):
    src = edge_index[0]
    dst = edge_index[1]
    deg2 = _sc_degrees(src, dst)
    out_deg = deg2[0, :, 0:1]
    in_deg = deg2[1, :, 0:1]

    x, norm_src, norm_dst = _tc_prep(h, out_deg, in_deg)

    for (w, b) in ((W0, b0), (W1, b1), (W2, b2)):
        agg3 = _sc_aggregate(x.reshape(2 * N_NODES, HD), src, dst)
        x = _tc_layer(agg3, norm_dst, norm_src, w, b)

    agg3 = _sc_aggregate(x.reshape(2 * N_NODES, HD), src, dst)
    return _tc_final(agg3, norm_dst, W3, b3, fc1_W, fc1_b, fc2_W, fc2_b)


# speedup vs baseline: 1.0686x; 1.0572x over previous
import functools

import jax
import jax.numpy as jnp
from jax import lax
from jax.experimental import pallas as pl
from jax.experimental.pallas import tpu as pltpu
from jax.experimental.pallas import tpu_sc as plsc

N_NODES = 10000
N_EDGES = 160000
D = 256
HD = 128
N_CLS = 10

NC = 2
NS = 16
LANES = 16

EDGES_PER_TILE = N_EDGES // NS
CHUNK = 128
N_FULL = EDGES_PER_TILE // CHUNK
REM = EDGES_PER_TILE - N_FULL * CHUNK
ROWS_PER_TILE = N_NODES // NS

HALF = EDGES_PER_TILE // 2
HCH = HALF // CHUNK
HREM = HALF - HCH * CHUNK
ACC_ROWS = N_NODES + 8

_sc_mesh = plsc.VectorSubcoreMesh(
    core_axis_name="c", subcore_axis_name="s", num_cores=NC, num_subcores=NS)


RO_ROWS = 624
RO_LAST = N_NODES - RO_ROWS * (NS - 1)


def _readout(shared_ref, out_hbm, c, s):
    row8 = pl.multiple_of(s * RO_ROWS, 8)

    @pl.when(s < NS - 1)
    def _():
        pltpu.sync_copy(shared_ref.at[pl.ds(row8, RO_ROWS)],
                        out_hbm.at[c, pl.ds(row8, RO_ROWS)])

    @pl.when(s == NS - 1)
    def _():
        pltpu.sync_copy(shared_ref.at[pl.ds(row8, RO_LAST)],
                        out_hbm.at[c, pl.ds(row8, RO_LAST)])


def _zero_vmem_rows(ref, nrows, ncols):
    zv = jnp.zeros((LANES,), jnp.float32)
    for j in range(ncols // LANES):
        def body(i, _, j=j):
            ref[i, pl.ds(j * LANES, LANES)] = zv
            return 0
        lax.fori_loop(0, nrows, body, 0)


@functools.partial(
    pl.kernel,
    out_type=jax.ShapeDtypeStruct((NC, N_NODES, HD), jnp.float32),
    mesh=_sc_mesh,
    scratch_types=[
        pltpu.VMEM_SHARED((N_NODES, HD), jnp.float32),
        pltpu.VMEM((CHUNK, HD), jnp.float32),
        pltpu.VMEM((EDGES_PER_TILE,), jnp.int32),
        pltpu.VMEM((N_FULL, CHUNK), jnp.int32),
        pltpu.VMEM((REM,), jnp.int32),
        pltpu.SemaphoreType.DMA,
    ],
)
def _sc_degrees(src_hbm, dst_hbm, deg_hbm,
                hist, ones_v, draw, dst2d, didx_t, sem):
    c = lax.axis_index("c")
    s = lax.axis_index("s")

    _zero_vmem_rows(ones_v, 125, HD)
    row0 = s * ROWS_PER_TILE
    for k in range(5):
        pltpu.sync_copy(ones_v.at[pl.ds(0, 125)],
                        hist.at[pl.ds(row0 + k * 125, 125)])

    onev = jnp.ones((LANES,), jnp.float32)

    def fill_ones(i, _):
        for j in range(HD // LANES):
            ones_v[i, pl.ds(j * LANES, LANES)] = onev
        return 0
    lax.fori_loop(0, CHUNK, fill_ones, 0)

    base0 = s * EDGES_PER_TILE

    @pl.when(c == 0)
    def _():
        pltpu.sync_copy(src_hbm.at[pl.ds(base0, EDGES_PER_TILE)], draw)

    @pl.when(c == 1)
    def _():
        pltpu.sync_copy(dst_hbm.at[pl.ds(base0, EDGES_PER_TILE)], draw)

    def dfill(r, _):
        for j in range(HD // LANES):
            dst2d[r, pl.ds(j * LANES, LANES)] = \
                draw[pl.ds(r * CHUNK + j * LANES, LANES)]
        return 0
    lax.fori_loop(0, N_FULL, dfill, 0)
    didx_t[pl.ds(0, REM)] = draw[pl.ds(N_FULL * CHUNK, REM)]
    plsc.subcore_barrier()

    def fire6(r6, _):
        ds_ = [pltpu.async_copy(ones_v, hist.at[dst2d.at[r6 * 6 + j]], sem,
                                add=True)
               for j in range(6)]
        for d in ds_:
            d.wait()
        return 0
    lax.fori_loop(0, N_FULL // 6, fire6, 0)
    pltpu.sync_copy(ones_v.at[pl.ds(0, REM)], hist.at[didx_t], add=True)

    plsc.subcore_barrier()
    _readout(hist, deg_hbm, c, s)


@functools.partial(
    pl.kernel,
    out_type=jax.ShapeDtypeStruct((NC, N_NODES, HD), jnp.float32),
    mesh=_sc_mesh,
    scratch_types=[
        pltpu.VMEM_SHARED((ACC_ROWS, HD), jnp.float32),
        pltpu.VMEM((CHUNK, HD), jnp.float32),
        pltpu.VMEM((CHUNK, HD), jnp.float32),
        pltpu.VMEM((HALF + 8,), jnp.int32),
        pltpu.VMEM((HALF + 8,), jnp.int32),
        pltpu.VMEM((HCH, CHUNK), jnp.int32),
        pltpu.VMEM((LANES,), jnp.int32),
        pltpu.SemaphoreType.DMA,
        pltpu.SemaphoreType.DMA,
    ],
)
def _sc_aggregate(x2_hbm, src_hbm, dst_hbm, agg_hbm,
                  acc, rows_a, rows_b, sidx, draw, dst2d, didx_t,
                  sem_a, sem_b):
    c = lax.axis_index("c")
    s = lax.axis_index("s")
    lane = lax.iota(jnp.int32, LANES)
    row0 = s * ROWS_PER_TILE

    def gather(r, buf, sem):
        pltpu.async_copy(
            x2_hbm.at[sidx.at[pl.ds(r * CHUNK, CHUNK)]], buf, sem)

    def gwait(r, buf, sem):
        pltpu.make_async_copy(
            x2_hbm.at[sidx.at[pl.ds(r * CHUNK, CHUNK)]], buf, sem).wait()

    def scatter(r, buf):
        pltpu.sync_copy(buf, acc.at[dst2d.at[r]], add=True)

    def stage(h):
        base = s * EDGES_PER_TILE + h * HALF
        pltpu.sync_copy(src_hbm.at[pl.ds(base, HALF)], sidx.at[pl.ds(0, HALF)])
        pltpu.sync_copy(dst_hbm.at[pl.ds(base, HALF)], draw.at[pl.ds(0, HALF)])

        cbase = c * N_NODES

        def xform(i, _):
            v = sidx[pl.ds(i * LANES, LANES)]
            sidx[pl.ds(i * LANES, LANES)] = v + cbase
            return 0
        lax.fori_loop(0, HALF // LANES, xform, 0)
        v = sidx[pl.ds(HCH * CHUNK, LANES)]
        sidx[pl.ds(HCH * CHUNK, LANES)] = \
            jnp.where(lane < HREM, v + cbase, 0)
        d = draw[pl.ds(HCH * CHUNK, LANES)]
        didx_t[pl.ds(0, LANES)] = jnp.where(lane < HREM, d, N_NODES)

        def dfill(r, _):
            for j in range(CHUNK // LANES):
                dst2d[r, pl.ds(j * LANES, LANES)] = \
                    draw[pl.ds(r * CHUNK + j * LANES, LANES)]
            return 0
        lax.fori_loop(0, HCH, dfill, 0)

    def run_half():
        pltpu.async_copy(x2_hbm.at[sidx.at[pl.ds(HCH * CHUNK, LANES)]],
                         rows_a.at[pl.ds(0, LANES)], sem_a).wait()
        pltpu.sync_copy(rows_a.at[pl.ds(0, LANES)], acc.at[didx_t], add=True)
        gather(0, rows_a, sem_a)
        gwait(0, rows_a, sem_a)
        scatter(0, rows_a)

        gather(1, rows_a, sem_a)

        def body(r2, _):
            e = r2 * 2 + 1
            gwait(e, rows_a, sem_a)
            gather(e + 1, rows_b, sem_b)
            scatter(e, rows_a)
            gwait(e + 1, rows_b, sem_b)
            @pl.when(r2 < (HCH - 1) // 2 - 1)
            def _():
                gather(e + 2, rows_a, sem_a)
            scatter(e + 1, rows_b)
            return 0
        lax.fori_loop(0, (HCH - 1) // 2, body, 0)

    _zero_vmem_rows(rows_a, 125, HD)
    for k in range(5):
        pltpu.async_copy(rows_a.at[pl.ds(0, 125)],
                         acc.at[pl.ds(row0 + k * 125, 125)], sem_b)
    stage(0)
    for k in range(5):
        pltpu.make_async_copy(rows_a.at[pl.ds(0, 125)],
                              acc.at[pl.ds(row0 + k * 125, 125)],
                              sem_b).wait()
    plsc.subcore_barrier()

    run_half()
    stage(1)
    run_half()

    plsc.subcore_barrier()
    _readout(acc, agg_hbm, c, s)


ROWS_BLK = 1000
N_BLK = N_NODES // ROWS_BLK


def _elu(x):
    return jnp.where(x > 0, x, jnp.exp(x) - 1.0)


def _tc_prep_body(h_ref, od_ref, id_ref, x0_ref, ns_ref, nd_ref):
    od = od_ref[...]
    idg = id_ref[...]
    ns = lax.rsqrt(jnp.maximum(od, 1.0))
    nd = lax.rsqrt(jnp.maximum(idg, 1.0))
    ns_ref[...] = ns
    nd_ref[...] = nd
    x0 = h_ref[...] * ns
    x0_ref[0] = x0[:, 0:HD]
    x0_ref[1] = x0[:, HD:D]


def _tc_prep(h, out_deg, in_deg):
    return pl.pallas_call(
        _tc_prep_body,
        grid=(N_BLK,),
        in_specs=[
            pl.BlockSpec((ROWS_BLK, D), lambda i: (i, 0)),
            pl.BlockSpec((ROWS_BLK, 1), lambda i: (i, 0)),
            pl.BlockSpec((ROWS_BLK, 1), lambda i: (i, 0)),
        ],
        out_specs=[
            pl.BlockSpec((NC, ROWS_BLK, HD), lambda i: (0, i, 0)),
            pl.BlockSpec((ROWS_BLK, 1), lambda i: (i, 0)),
            pl.BlockSpec((ROWS_BLK, 1), lambda i: (i, 0)),
        ],
        out_shape=[
            jax.ShapeDtypeStruct((NC, N_NODES, HD), jnp.float32),
            jax.ShapeDtypeStruct((N_NODES, 1), jnp.float32),
            jax.ShapeDtypeStruct((N_NODES, 1), jnp.float32),
        ],
    )(h, out_deg, in_deg)


def _tc_layer_body(lo_ref, hi_ref, nd_ref, ns_ref, w_ref, b_ref, out_ref):
    nd = nd_ref[...]
    a_lo = lo_ref[0] * nd
    a_hi = hi_ref[0] * nd
    y = (jnp.dot(a_lo, w_ref[0:HD, :], preferred_element_type=jnp.float32)
         + jnp.dot(a_hi, w_ref[HD:D, :], preferred_element_type=jnp.float32)
         + b_ref[...])
    xn = _elu(y) * ns_ref[...]
    out_ref[0] = xn[:, 0:HD]
    out_ref[1] = xn[:, HD:D]


def _tc_layer(agg3, norm_dst, norm_src, w, b):
    return pl.pallas_call(
        _tc_layer_body,
        grid=(N_BLK,),
        in_specs=[
            pl.BlockSpec((1, ROWS_BLK, HD), lambda i: (0, i, 0)),
            pl.BlockSpec((1, ROWS_BLK, HD), lambda i: (1, i, 0)),
            pl.BlockSpec((ROWS_BLK, 1), lambda i: (i, 0)),
            pl.BlockSpec((ROWS_BLK, 1), lambda i: (i, 0)),
            pl.BlockSpec((D, D), lambda i: (0, 0)),
            pl.BlockSpec((1, D), lambda i: (0, 0)),
        ],
        out_specs=pl.BlockSpec((NC, ROWS_BLK, HD), lambda i: (0, i, 0)),
        out_shape=jax.ShapeDtypeStruct((NC, N_NODES, HD), jnp.float32),
    )(agg3, agg3, norm_dst, norm_src, w, b.reshape(1, D))


def _tc_final_body(lo_ref, hi_ref, nd_ref, w_ref, b_ref,
                   f1w_ref, f1b_ref, f2w_ref, f2b_ref, out_ref, acc_ref):
    i = pl.program_id(0)
    nd = nd_ref[...]
    a_lo = lo_ref[0] * nd
    a_hi = hi_ref[0] * nd
    y = (jnp.dot(a_lo, w_ref[0:HD, :], preferred_element_type=jnp.float32)
         + jnp.dot(a_hi, w_ref[HD:D, :], preferred_element_type=jnp.float32)
         + b_ref[...])
    y = _elu(y)
    blk = jnp.sum(y, axis=0, keepdims=True)

    @pl.when(i == 0)
    def _():
        acc_ref[...] = blk

    @pl.when(i > 0)
    def _():
        acc_ref[...] = acc_ref[...] + blk

    @pl.when(i == N_BLK - 1)
    def _():
        g = acc_ref[...] * (1.0 / N_NODES)
        g1 = _elu(jnp.dot(g, f1w_ref[...], preferred_element_type=jnp.float32)
                  + f1b_ref[...])
        g2 = (jnp.dot(g1, f2w_ref[...], preferred_element_type=jnp.float32)
              + f2b_ref[...])
        m = jnp.max(g2, axis=1, keepdims=True)
        lse = m + jnp.log(jnp.sum(jnp.exp(g2 - m), axis=1, keepdims=True))
        out_ref[...] = g2 - lse


def _tc_final(agg3, norm_dst, w, b, f1w, f1b, f2w, f2b):
    return pl.pallas_call(
        _tc_final_body,
        grid=(N_BLK,),
        in_specs=[
            pl.BlockSpec((1, ROWS_BLK, HD), lambda i: (0, i, 0)),
            pl.BlockSpec((1, ROWS_BLK, HD), lambda i: (1, i, 0)),
            pl.BlockSpec((ROWS_BLK, 1), lambda i: (i, 0)),
            pl.BlockSpec((D, D), lambda i: (0, 0)),
            pl.BlockSpec((1, D), lambda i: (0, 0)),
            pl.BlockSpec((D, D), lambda i: (0, 0)),
            pl.BlockSpec((1, D), lambda i: (0, 0)),
            pl.BlockSpec((D, N_CLS), lambda i: (0, 0)),
            pl.BlockSpec((1, N_CLS), lambda i: (0, 0)),
        ],
        out_specs=pl.BlockSpec((1, N_CLS), lambda i: (0, 0)),
        out_shape=jax.ShapeDtypeStruct((1, N_CLS), jnp.float32),
        scratch_shapes=[pltpu.VMEM((1, D), jnp.float32)],
    )(agg3, agg3, norm_dst, w, b.reshape(1, D),
      f1w, f1b.reshape(1, D), f2w, f2b.reshape(1, N_CLS))


def kernel(h, edge_index, W0, b0, W1, b1, W2, b2, W3, b3,
           fc1_W, fc1_b, fc2_W, fc2_b):
    src = edge_index[0]
    dst = edge_index[1]
    deg2 = _sc_degrees(src, dst)
    out_deg = deg2[0, :, 0:1]
    in_deg = deg2[1, :, 0:1]

    x, norm_src, norm_dst = _tc_prep(h, out_deg, in_deg)

    for (w, b) in ((W0, b0), (W1, b1), (W2, b2)):
        agg3 = _sc_aggregate(x.reshape(NC * N_NODES, HD), src, dst)
        x = _tc_layer(agg3, norm_dst, norm_src, w, b)

    agg3 = _sc_aggregate(x.reshape(NC * N_NODES, HD), src, dst)
    return _tc_final(agg3, norm_dst, W3, b3, fc1_W, fc1_b, fc2_W, fc2_b)
